# baseline (speedup 1.0000x reference)
import jax
import jax.numpy as jnp
import numpy as np
from jax.experimental import pallas as pl
from jax.experimental.pallas import tpu as pltpu
from jax.experimental.shard_map import shard_map

N = 2400
TQ = 480
NT = N // TQ
K = 10
NEG_INF = float("-inf")


def _masked_attention(e, nb_mask, h):
    e = jnp.where(nb_mask, e, NEG_INF)
    mx = jnp.max(e, axis=1, keepdims=True)
    p = jnp.exp(e - mx)
    agg = jnp.dot(p, h, preferred_element_type=jnp.float32)
    return agg / jnp.sum(p, axis=1, keepdims=True)


def _main_kernel(xyz_ref, g1w_ref, g1src_ref, g1dst_ref, g1res_ref, g1b_ref,
                 g2w_ref, g2src_ref, g2dst_ref, g2res_ref, g2b_ref,
                 cw_ref, cb_ref, cw2_ref, vlad_ref, msk_s, g1_s, g2_s):
    xyz = xyz_ref[0]
    iota = jax.lax.broadcasted_iota(jnp.int32, (TQ, N), 1)

    sq_row = jax.lax.dot_general(jnp.ones((1, 3), jnp.float32), xyz * xyz,
                                 (((1,), (1,)), ((), ())),
                                 preferred_element_type=jnp.float32)
    h1 = jnp.dot(xyz, g1w_ref[...], preferred_element_type=jnp.float32)
    s_dst = jax.lax.dot_general(g1dst_ref[...], h1, (((1,), (1,)), ((), ())),
                                preferred_element_type=jnp.float32)

    def sel_body(i, _):
        q0 = i * TQ
        xq = xyz_ref[0, pl.ds(q0, TQ), :]
        dot = jax.lax.dot_general(xq, xyz, (((1,), (1,)), ((), ())),
                                  preferred_element_type=jnp.float32)
        d = jnp.maximum(jnp.sum(xq * xq, axis=1, keepdims=True)
                        + sq_row - 2.0 * dot, 0.0)
        for _ in range(K):
            rowmin = jnp.min(d, axis=1, keepdims=True)
            idx = jnp.min(jnp.where(d == rowmin, iota, jnp.int32(N)),
                          axis=1, keepdims=True)
            d = jnp.where(iota == idx, jnp.inf, d)
        nb = d == jnp.inf
        msk_s[pl.ds(q0, TQ), :] = jnp.where(
            nb, 1.0, 0.0).astype(jnp.bfloat16)

        hq = jnp.dot(xq, g1w_ref[...], preferred_element_type=jnp.float32)
        s_src = jax.lax.dot_general(hq, g1src_ref[...],
                                    (((1,), (1,)), ((), ())),
                                    preferred_element_type=jnp.float32)
        e = s_src + s_dst
        e = jnp.where(e >= 0, e, 0.2 * e)
        agg = _masked_attention(e, nb, h1)
        res = jnp.dot(xq, g1res_ref[...],
                      preferred_element_type=jnp.float32) + g1b_ref[...]
        g1_s[pl.ds(q0, TQ), :] = jnp.maximum(agg + res, 0.0)
        return 0

    jax.lax.fori_loop(0, NT, sel_body, 0)

    g1 = g1_s[...]
    h2 = jnp.dot(g1, g2w_ref[...], preferred_element_type=jnp.float32)
    s_dst2 = jax.lax.dot_general(g2dst_ref[...], h2, (((1,), (1,)), ((), ())),
                                 preferred_element_type=jnp.float32)

    def gat2_body(i, _):
        q0 = i * TQ
        gq = g1_s[pl.ds(q0, TQ), :]
        hq = jnp.dot(gq, g2w_ref[...], preferred_element_type=jnp.float32)
        s_src = jax.lax.dot_general(hq, g2src_ref[...],
                                    (((1,), (1,)), ((), ())),
                                    preferred_element_type=jnp.float32)
        e = s_src + s_dst2
        e = jnp.where(e >= 0, e, 0.2 * e)
        nb = msk_s[pl.ds(q0, TQ), :].astype(jnp.float32) > 0
        agg = _masked_attention(e, nb, h2)
        res = jnp.dot(gq, g2res_ref[...],
                      preferred_element_type=jnp.float32) + g2b_ref[...]
        g2_s[pl.ds(q0, TQ), :] = agg + res
        return 0

    jax.lax.fori_loop(0, NT, gat2_body, 0)

    g2 = g2_s[...]
    act = jnp.dot(g2, cw_ref[...], preferred_element_type=jnp.float32) + cb_ref[...]
    am = jnp.max(act, axis=1, keepdims=True)
    ap = jnp.exp(act - am)
    act = ap / jnp.sum(ap, axis=1, keepdims=True)
    a_sum = jnp.sum(act, axis=0, keepdims=True)
    vlad = jax.lax.dot_general(g2, act, (((0,), (0,)), ((), ())),
                               preferred_element_type=jnp.float32)
    vlad = vlad - a_sum * cw2_ref[...]
    nrm = jnp.sqrt(jnp.sum(vlad * vlad, axis=0, keepdims=True))
    vlad_ref[0] = vlad / (nrm + 1e-12)


def _head_kernel(v_ref, hw_ref, hb_ref, gw_ref, gb_ref, out_ref):
    v = v_ref[...]
    nrm = jnp.sqrt(jnp.sum(v * v, axis=1, keepdims=True))
    v = v / (nrm + 1e-12)
    out = jnp.dot(v, hw_ref[...], preferred_element_type=jnp.float32) + hb_ref[...]
    gates = jnp.dot(out, gw_ref[...], preferred_element_type=jnp.float32) + gb_ref[...]
    gates = 1.0 / (1.0 + jnp.exp(-gates))
    out_ref[...] = out * gates


def _pipeline_local(xyz, gat1_W, gat1_asrc, gat1_adst, gat1_res, gat1_b,
                    gat2_W, gat2_asrc, gat2_adst, gat2_res, gat2_b,
                    cluster_w, cluster_b, cluster_w2, hidden_w, hidden_b,
                    gating_w, gating_b):
    Bl = xyz.shape[0]
    batch_spec = pl.BlockSpec((1, N, 3), lambda b: (b, 0, 0))
    full = lambda s: pl.BlockSpec(s, lambda b: tuple(0 for _ in s))

    vlad = pl.pallas_call(
        _main_kernel,
        grid=(Bl,),
        in_specs=[
            batch_spec,
            full((3, 128)), full((1, 128)), full((1, 128)), full((3, 128)),
            full((1, 128)),
            full((128, 256)), full((1, 256)), full((1, 256)), full((128, 256)),
            full((1, 256)),
            full((256, 64)), full((1, 64)), full((256, 64)),
        ],
        out_specs=pl.BlockSpec((1, 256, 64), lambda b: (b, 0, 0)),
        out_shape=jax.ShapeDtypeStruct((Bl, 256, 64), jnp.float32),
        scratch_shapes=[pltpu.VMEM((N, N), jnp.bfloat16),
                        pltpu.VMEM((N, 128), jnp.float32),
                        pltpu.VMEM((N, 256), jnp.float32)],
        compiler_params=pltpu.CompilerParams(
            dimension_semantics=("parallel",)),
    )(xyz, gat1_W, gat1_asrc, gat1_adst, gat1_res, gat1_b,
      gat2_W, gat2_asrc, gat2_adst, gat2_res, gat2_b,
      cluster_w, cluster_b, cluster_w2)

    vflat = vlad.reshape(Bl, 256 * 64)
    return pl.pallas_call(
        _head_kernel,
        in_specs=[pl.BlockSpec(vflat.shape, lambda: (0, 0)),
                  pl.BlockSpec(hidden_w.shape, lambda: (0, 0)),
                  pl.BlockSpec((1, 256), lambda: (0, 0)),
                  pl.BlockSpec(gating_w.shape, lambda: (0, 0)),
                  pl.BlockSpec((1, 256), lambda: (0, 0))],
        out_specs=pl.BlockSpec((Bl, 256), lambda: (0, 0)),
        out_shape=jax.ShapeDtypeStruct((Bl, 256), jnp.float32),
    )(vflat, hidden_w, hidden_b, gating_w, gating_b)


@jax.jit
def kernel(x, depth_map, conv1_w, conv1_b, conv2_w, conv2_b, conv3_w, conv3_b,
           gat1_W, gat1_asrc, gat1_adst, gat1_res, gat1_b,
           gat2_W, gat2_asrc, gat2_adst, gat2_res, gat2_b,
           cluster_w, cluster_b, cluster_w2, hidden_w, hidden_b,
           gating_w, gating_b):
    B, hh, ww = depth_map.shape
    theta = jnp.linspace(-jnp.pi, jnp.pi, ww)
    phi = jnp.linspace(-jnp.pi / 2, jnp.pi / 2, hh)
    r = depth_map
    cx = r * jnp.cos(phi[None, :, None]) * jnp.sin(theta[None, None, :])
    cy = r * jnp.sin(phi[None, :, None])
    cz = r * jnp.cos(phi[None, :, None]) * jnp.cos(theta[None, None, :])
    xyz = jnp.stack((cx, cy, cz), axis=-1).reshape(B, -1, 3)

    args = (xyz, gat1_W, gat1_asrc[None, :], gat1_adst[None, :], gat1_res,
            gat1_b[None, :],
            gat2_W, gat2_asrc[None, :], gat2_adst[None, :], gat2_res,
            gat2_b[None, :],
            cluster_w, cluster_b[None, :], cluster_w2[0],
            hidden_w, hidden_b[None, :], gating_w, gating_b[None, :])

    devs = jax.devices()
    nd = max(n for n in range(1, len(devs) + 1) if B % n == 0)
    mesh = jax.sharding.Mesh(np.array(devs[:nd]), ("d",))
    P = jax.sharding.PartitionSpec
    in_specs = (P("d"),) + (P(),) * (len(args) - 1)
    fn = shard_map(_pipeline_local, mesh=mesh, in_specs=in_specs,
                   out_specs=P("d"), check_rep=False)
    return fn(*args)


# speedup vs baseline: 2.4987x; 2.4987x over previous
import jax
import jax.numpy as jnp
from jax.experimental import pallas as pl
from jax.experimental.pallas import tpu as pltpu

N = 2400
TQ = 480
NT = N // TQ
K = 10
NEG_INF = float("-inf")


def _masked_attention(e, nb_mask, h):
    e = jnp.where(nb_mask, e, NEG_INF)
    mx = jnp.max(e, axis=1, keepdims=True)
    p = jnp.exp(e - mx)
    agg = jnp.dot(p, h, preferred_element_type=jnp.float32)
    return agg / jnp.sum(p, axis=1, keepdims=True)


def _main_kernel(xyz_ref, g1w_ref, g1src_ref, g1dst_ref, g1res_ref, g1b_ref,
                 g2w_ref, g2src_ref, g2dst_ref, g2res_ref, g2b_ref,
                 cw_ref, cb_ref, cw2_ref, vlad_ref, msk_s, g1_s, g2_s):
    xyz = xyz_ref[0]
    iota = jax.lax.broadcasted_iota(jnp.int32, (TQ, N), 1)

    sq_row = jax.lax.dot_general(jnp.ones((1, 3), jnp.float32), xyz * xyz,
                                 (((1,), (1,)), ((), ())),
                                 preferred_element_type=jnp.float32)
    h1 = jnp.dot(xyz, g1w_ref[...], preferred_element_type=jnp.float32)
    s_dst = jax.lax.dot_general(g1dst_ref[...], h1, (((1,), (1,)), ((), ())),
                                preferred_element_type=jnp.float32)

    def sel_body(i, _):
        q0 = i * TQ
        xq = xyz_ref[0, pl.ds(q0, TQ), :]
        dot = jax.lax.dot_general(xq, xyz, (((1,), (1,)), ((), ())),
                                  preferred_element_type=jnp.float32)
        d = jnp.maximum(jnp.sum(xq * xq, axis=1, keepdims=True)
                        + sq_row - 2.0 * dot, 0.0)
        for _ in range(K):
            am = jax.lax.argmin(d, 1, jnp.int32)
            d = jnp.where(iota == am[:, None], jnp.inf, d)
        nb = d == jnp.inf
        msk_s[pl.ds(q0, TQ), :] = jnp.where(
            nb, 1.0, 0.0).astype(jnp.bfloat16)

        hq = jnp.dot(xq, g1w_ref[...], preferred_element_type=jnp.float32)
        s_src = jax.lax.dot_general(hq, g1src_ref[...],
                                    (((1,), (1,)), ((), ())),
                                    preferred_element_type=jnp.float32)
        e = s_src + s_dst
        e = jnp.where(e >= 0, e, 0.2 * e)
        agg = _masked_attention(e, nb, h1)
        res = jnp.dot(xq, g1res_ref[...],
                      preferred_element_type=jnp.float32) + g1b_ref[...]
        g1_s[pl.ds(q0, TQ), :] = jnp.maximum(agg + res, 0.0)
        return 0

    jax.lax.fori_loop(0, NT, sel_body, 0)

    g1 = g1_s[...]
    h2 = jnp.dot(g1, g2w_ref[...], preferred_element_type=jnp.float32)
    s_dst2 = jax.lax.dot_general(g2dst_ref[...], h2, (((1,), (1,)), ((), ())),
                                 preferred_element_type=jnp.float32)

    def gat2_body(i, _):
        q0 = i * TQ
        gq = g1_s[pl.ds(q0, TQ), :]
        hq = jnp.dot(gq, g2w_ref[...], preferred_element_type=jnp.float32)
        s_src = jax.lax.dot_general(hq, g2src_ref[...],
                                    (((1,), (1,)), ((), ())),
                                    preferred_element_type=jnp.float32)
        e = s_src + s_dst2
        e = jnp.where(e >= 0, e, 0.2 * e)
        nb = msk_s[pl.ds(q0, TQ), :].astype(jnp.float32) > 0
        agg = _masked_attention(e, nb, h2)
        res = jnp.dot(gq, g2res_ref[...],
                      preferred_element_type=jnp.float32) + g2b_ref[...]
        g2_s[pl.ds(q0, TQ), :] = agg + res
        return 0

    jax.lax.fori_loop(0, NT, gat2_body, 0)

    g2 = g2_s[...]
    act = jnp.dot(g2, cw_ref[...], preferred_element_type=jnp.float32) + cb_ref[...]
    am = jnp.max(act, axis=1, keepdims=True)
    ap = jnp.exp(act - am)
    act = ap / jnp.sum(ap, axis=1, keepdims=True)
    a_sum = jnp.sum(act, axis=0, keepdims=True)
    vlad = jax.lax.dot_general(g2, act, (((0,), (0,)), ((), ())),
                               preferred_element_type=jnp.float32)
    vlad = vlad - a_sum * cw2_ref[...]
    nrm = jnp.sqrt(jnp.sum(vlad * vlad, axis=0, keepdims=True))
    vlad_ref[0] = vlad / (nrm + 1e-12)


def _head_kernel(v_ref, hw_ref, hb_ref, gw_ref, gb_ref, out_ref):
    v = v_ref[...]
    nrm = jnp.sqrt(jnp.sum(v * v, axis=1, keepdims=True))
    v = v / (nrm + 1e-12)
    out = jnp.dot(v, hw_ref[...], preferred_element_type=jnp.float32) + hb_ref[...]
    gates = jnp.dot(out, gw_ref[...], preferred_element_type=jnp.float32) + gb_ref[...]
    gates = 1.0 / (1.0 + jnp.exp(-gates))
    out_ref[...] = out * gates


def _pipeline_local(xyz, gat1_W, gat1_asrc, gat1_adst, gat1_res, gat1_b,
                    gat2_W, gat2_asrc, gat2_adst, gat2_res, gat2_b,
                    cluster_w, cluster_b, cluster_w2, hidden_w, hidden_b,
                    gating_w, gating_b):
    Bl = xyz.shape[0]
    batch_spec = pl.BlockSpec((1, N, 3), lambda b: (b, 0, 0))
    full = lambda s: pl.BlockSpec(s, lambda b: tuple(0 for _ in s))

    vlad = pl.pallas_call(
        _main_kernel,
        grid=(Bl,),
        in_specs=[
            batch_spec,
            full((3, 128)), full((1, 128)), full((1, 128)), full((3, 128)),
            full((1, 128)),
            full((128, 256)), full((1, 256)), full((1, 256)), full((128, 256)),
            full((1, 256)),
            full((256, 64)), full((1, 64)), full((256, 64)),
        ],
        out_specs=pl.BlockSpec((1, 256, 64), lambda b: (b, 0, 0)),
        out_shape=jax.ShapeDtypeStruct((Bl, 256, 64), jnp.float32),
        scratch_shapes=[pltpu.VMEM((N, N), jnp.bfloat16),
                        pltpu.VMEM((N, 128), jnp.float32),
                        pltpu.VMEM((N, 256), jnp.float32)],
        compiler_params=pltpu.CompilerParams(
            dimension_semantics=("parallel",)),
    )(xyz, gat1_W, gat1_asrc, gat1_adst, gat1_res, gat1_b,
      gat2_W, gat2_asrc, gat2_adst, gat2_res, gat2_b,
      cluster_w, cluster_b, cluster_w2)

    vflat = vlad.reshape(Bl, 256 * 64)
    return pl.pallas_call(
        _head_kernel,
        in_specs=[pl.BlockSpec(vflat.shape, lambda: (0, 0)),
                  pl.BlockSpec(hidden_w.shape, lambda: (0, 0)),
                  pl.BlockSpec((1, 256), lambda: (0, 0)),
                  pl.BlockSpec(gating_w.shape, lambda: (0, 0)),
                  pl.BlockSpec((1, 256), lambda: (0, 0))],
        out_specs=pl.BlockSpec((Bl, 256), lambda: (0, 0)),
        out_shape=jax.ShapeDtypeStruct((Bl, 256), jnp.float32),
    )(vflat, hidden_w, hidden_b, gating_w, gating_b)


@jax.jit
def kernel(x, depth_map, conv1_w, conv1_b, conv2_w, conv2_b, conv3_w, conv3_b,
           gat1_W, gat1_asrc, gat1_adst, gat1_res, gat1_b,
           gat2_W, gat2_asrc, gat2_adst, gat2_res, gat2_b,
           cluster_w, cluster_b, cluster_w2, hidden_w, hidden_b,
           gating_w, gating_b):
    B, hh, ww = depth_map.shape
    theta = jnp.linspace(-jnp.pi, jnp.pi, ww)
    phi = jnp.linspace(-jnp.pi / 2, jnp.pi / 2, hh)
    r = depth_map
    cx = r * jnp.cos(phi[None, :, None]) * jnp.sin(theta[None, None, :])
    cy = r * jnp.sin(phi[None, :, None])
    cz = r * jnp.cos(phi[None, :, None]) * jnp.cos(theta[None, None, :])
    xyz = jnp.stack((cx, cy, cz), axis=-1).reshape(B, -1, 3)

    args = (xyz, gat1_W, gat1_asrc[None, :], gat1_adst[None, :], gat1_res,
            gat1_b[None, :],
            gat2_W, gat2_asrc[None, :], gat2_adst[None, :], gat2_res,
            gat2_b[None, :],
            cluster_w, cluster_b[None, :], cluster_w2[0],
            hidden_w, hidden_b[None, :], gating_w, gating_b[None, :])

    return _pipeline_local(*args)
